# Initial kernel scaffold; baseline (speedup 1.0000x reference)
import jax
import jax.numpy as jnp
from jax.experimental import pallas as pl


def kernel(x, W_np, b_np, Wq, Wk, Wv, Wo, Wh, bh):
    raise NotImplementedError("write your pallas kernel here")


# speedup vs baseline: 2.0058x; 2.0058x over previous
import functools

import jax
import jax.numpy as jnp
from jax.experimental import pallas as pl
from jax.experimental.pallas import tpu as pltpu

_LAYERS = 4
_M = 72
_KN = 32
_DP = 32
_DL = 8
_DH = 64
_KH = 4
_KA = 16
_NC = 40

_BIG = 1e30
_NEG = -1e30


def _bf(a):
    return a.astype(jnp.bfloat16)


def _bdot(a, b):
    return jnp.dot(_bf(a), _bf(b), preferred_element_type=jnp.float32)


def _fps_body(x0r, x1r, x2r, kp0r, kp1r, kp2r):
    B, N = x0r.shape
    x0, x1, x2 = x0r[:], x1r[:], x2r[:]
    lanes = jax.lax.broadcasted_iota(jnp.int32, (B, N), 1)
    lanesM = jax.lax.broadcasted_iota(jnp.int32, (B, _M), 1)

    def step(t, carry):
        md, last, k0, k1, k2 = carry
        oh = (lanes == last).astype(jnp.float32)
        lp0 = jnp.sum(x0 * oh, axis=1, keepdims=True)
        lp1 = jnp.sum(x1 * oh, axis=1, keepdims=True)
        lp2 = jnp.sum(x2 * oh, axis=1, keepdims=True)
        slot = (lanesM == t).astype(jnp.float32)
        k0 = k0 + lp0 * slot
        k1 = k1 + lp1 * slot
        k2 = k2 + lp2 * slot
        d = (x0 - lp0) ** 2 + (x1 - lp1) ** 2 + (x2 - lp2) ** 2
        md = jnp.minimum(md, d)
        mx = jnp.max(md, axis=1, keepdims=True)
        cand = jnp.where(md == mx, lanes, N)
        nxt = jnp.min(cand, axis=1, keepdims=True).astype(jnp.int32)
        return md, nxt, k0, k1, k2

    md0 = jnp.full((B, N), 1e10, dtype=jnp.float32)
    last0 = jnp.zeros((B, 1), dtype=jnp.int32)
    kz = jnp.zeros((B, _M), dtype=jnp.float32)
    _, _, k0, k1, k2 = jax.lax.fori_loop(0, _M, step,
                                         (md0, last0, kz, kz, kz))
    kp0r[:] = k0
    kp1r[:] = k1
    kp2r[:] = k2


def _main_body(x0r, x1r, x2r, kp0r, kp1r, kp2r, wtr, bcr,
               wqr, wkr, wvr, wor, whr, bhr, outr, hscr):
    N = x0r.shape[-1]
    x0, x1, x2 = x0r[0], x1r[0], x2r[0]
    kp0, kp1, kp2 = kp0r[0], kp1r[0], kp2r[0]

    d = (kp0 - x0) ** 2 + (kp1 - x1) ** 2 + (kp2 - x2) ** 2

    wb = _bf(wtr[:]).astype(jnp.float32)
    xb0 = _bf(x0).astype(jnp.float32)
    xb1 = _bf(x1).astype(jnp.float32)
    xb2 = _bf(x2).astype(jnp.float32)
    fT = jnp.maximum(
        wb[:, 0:1] * xb0 + wb[:, 1:2] * xb1 + wb[:, 2:3] * xb2 + bcr[:],
        0.0)

    lanes = jax.lax.broadcasted_iota(jnp.int32, (_M, N), 1)

    def sel(_, carry):
        dd, mask = carry
        mn = jnp.min(dd, axis=1, keepdims=True)
        cand = jnp.where(dd == mn, lanes, N)
        si = jnp.min(cand, axis=1, keepdims=True)
        oh = lanes == si
        return jnp.where(oh, _BIG, dd), mask + oh.astype(jnp.float32)

    mask0 = jnp.zeros((_M, N), dtype=jnp.float32)
    _, maskf = jax.lax.fori_loop(0, _KN, sel, (d, mask0))
    mask = maskf > 0.0

    for c in range(_DP):
        fc = fT[c:c + 1, :]
        hscr[:, c:c + 1] = jnp.max(jnp.where(mask, fc, -1.0), axis=1,
                                   keepdims=True)
    h = hscr[:]

    lanesM = jax.lax.broadcasted_iota(jnp.int32, (_M, _M), 1)
    inv_sqrt = jnp.float32(1.0) / jnp.sqrt(jnp.float32(_DL))
    dv = _DH // _KH

    for l in range(_LAYERS):
        q = _bdot(h, wqr[l])
        k = _bdot(h, wkr[l])
        v = _bdot(h, wvr[l])
        acc = jnp.zeros((_M, _DP), dtype=jnp.float32)
        for hd in range(_KH):
            qh = q[:, hd * _DL:(hd + 1) * _DL]
            kh = k[:, hd * _DL:(hd + 1) * _DL]
            vh = v[:, hd * dv:(hd + 1) * dv]
            s = jax.lax.dot_general(
                _bf(qh), _bf(kh), (((1,), (1,)), ((), ())),
                preferred_element_type=jnp.float32) * inv_sqrt

            def asel(_, carry):
                sc, am = carry
                mx = jnp.max(sc, axis=1, keepdims=True)
                cand = jnp.where(sc == mx, lanesM, _M)
                si = jnp.min(cand, axis=1, keepdims=True)
                oh = lanesM == si
                return jnp.where(oh, _NEG, sc), am + oh.astype(jnp.float32)

            am0 = jnp.zeros((_M, _M), dtype=jnp.float32)
            _, amf = jax.lax.fori_loop(0, _KA, asel, (s, am0))

            sm = jnp.where(amf > 0.0, s, _NEG)
            mx = jnp.max(sm, axis=1, keepdims=True)
            e = jnp.exp(sm - mx)
            p = e / jnp.sum(e, axis=1, keepdims=True)
            oh_out = jnp.dot(p, vh, preferred_element_type=jnp.float32,
                             precision=jax.lax.Precision.HIGHEST)
            acc = acc + _bdot(oh_out, wor[l][hd * dv:(hd + 1) * dv, :])
        h = h + acc

    g = jnp.max(h, axis=0, keepdims=True)
    logits = _bdot(g, whr[:]) + bhr[:]
    z = logits - jnp.max(logits, axis=1, keepdims=True)
    e = jnp.exp(z)
    outr[:] = (e / jnp.sum(e, axis=1, keepdims=True))[None]


def kernel(x, W_np, b_np, Wq, Wk, Wv, Wo, Wh, bh):
    B, N, _ = x.shape
    f32 = jnp.float32
    x0 = x[:, :, 0]
    x1 = x[:, :, 1]
    x2 = x[:, :, 2]

    kp0, kp1, kp2 = pl.pallas_call(
        _fps_body,
        out_shape=[jax.ShapeDtypeStruct((B, _M), f32)] * 3,
    )(x0, x1, x2)

    x0r = x0.reshape(B, 1, N)
    x1r = x1.reshape(B, 1, N)
    x2r = x2.reshape(B, 1, N)
    kp0r = kp0.reshape(B, _M, 1)
    kp1r = kp1.reshape(B, _M, 1)
    kp2r = kp2.reshape(B, _M, 1)
    wt = W_np.T
    bc = b_np.reshape(_DP, 1)
    bhr = bh.reshape(1, _NC)

    row = lambda i: (i, 0, 0)
    full2 = lambda a: pl.BlockSpec(a.shape, lambda i: (0, 0))
    full3 = lambda a: pl.BlockSpec(a.shape, lambda i: (0, 0, 0))

    out = pl.pallas_call(
        _main_body,
        grid=(B,),
        in_specs=[
            pl.BlockSpec((1, 1, N), row), pl.BlockSpec((1, 1, N), row),
            pl.BlockSpec((1, 1, N), row),
            pl.BlockSpec((1, _M, 1), row), pl.BlockSpec((1, _M, 1), row),
            pl.BlockSpec((1, _M, 1), row),
            full2(wt), full2(bc),
            full3(Wq), full3(Wk), full3(Wv), full3(Wo),
            full2(Wh), full2(bhr),
        ],
        out_specs=pl.BlockSpec((1, 1, _NC), row),
        out_shape=jax.ShapeDtypeStruct((B, 1, _NC), f32),
        scratch_shapes=[pltpu.VMEM((_M, _DP), f32)],
    )(x0r, x1r, x2r, kp0r, kp1r, kp2r, wt, bc, Wq, Wk, Wv, Wo, Wh, bhr)

    return out.reshape(B, _NC)


# speedup vs baseline: 4.6604x; 2.3235x over previous
import functools

import jax
import jax.numpy as jnp
from jax.experimental import pallas as pl
from jax.experimental.pallas import tpu as pltpu

_LAYERS = 4
_M = 72
_KN = 32
_DP = 32
_DL = 8
_DH = 64
_KH = 4
_KA = 16
_NC = 40

_BIG = 1e30
_NEG = -1e30


def _bf(a):
    return a.astype(jnp.bfloat16)


def _bdot(a, b):
    return jnp.dot(_bf(a), _bf(b), preferred_element_type=jnp.float32)


def _fps_body(x0r, x1r, x2r, kp0r, kp1r, kp2r):
    B, N = x0r.shape
    x0, x1, x2 = x0r[:], x1r[:], x2r[:]
    lanes = jax.lax.broadcasted_iota(jnp.int32, (B, N), 1)
    lanesM = jax.lax.broadcasted_iota(jnp.int32, (B, _M), 1)

    def step(t, carry):
        md, last, k0, k1, k2 = carry
        oh = (lanes == last).astype(jnp.float32)
        lp0 = jnp.sum(x0 * oh, axis=1, keepdims=True)
        lp1 = jnp.sum(x1 * oh, axis=1, keepdims=True)
        lp2 = jnp.sum(x2 * oh, axis=1, keepdims=True)
        slot = (lanesM == t).astype(jnp.float32)
        k0 = k0 + lp0 * slot
        k1 = k1 + lp1 * slot
        k2 = k2 + lp2 * slot
        d = (x0 - lp0) ** 2 + (x1 - lp1) ** 2 + (x2 - lp2) ** 2
        md = jnp.minimum(md, d)
        mx = jnp.max(md, axis=1, keepdims=True)
        cand = jnp.where(md == mx, lanes, N)
        nxt = jnp.min(cand, axis=1, keepdims=True).astype(jnp.int32)
        return md, nxt, k0, k1, k2

    md0 = jnp.full((B, N), 1e10, dtype=jnp.float32)
    last0 = jnp.zeros((B, 1), dtype=jnp.int32)
    kz = jnp.zeros((B, _M), dtype=jnp.float32)
    _, _, k0, k1, k2 = jax.lax.fori_loop(0, _M, step,
                                         (md0, last0, kz, kz, kz))
    kp0r[:] = k0
    kp1r[:] = k1
    kp2r[:] = k2


def _main_body(x0r, x1r, x2r, kp0r, kp1r, kp2r, wtr, bcr,
               wqr, wkr, wvr, wor, whr, bhr, outr, hscr):
    N = x0r.shape[-1]
    x0, x1, x2 = x0r[0], x1r[0], x2r[0]
    kp0, kp1, kp2 = kp0r[0], kp1r[0], kp2r[0]

    d = (kp0 - x0) ** 2 + (kp1 - x1) ** 2 + (kp2 - x2) ** 2

    wb = _bf(wtr[:]).astype(jnp.float32)
    xb0 = _bf(x0).astype(jnp.float32)
    xb1 = _bf(x1).astype(jnp.float32)
    xb2 = _bf(x2).astype(jnp.float32)
    fT = jnp.maximum(
        wb[:, 0:1] * xb0 + wb[:, 1:2] * xb1 + wb[:, 2:3] * xb2 + bcr[:],
        0.0)

    lanes = jax.lax.broadcasted_iota(jnp.int32, (_M, N), 1)

    def sel(_, dd):
        mn = jnp.min(dd, axis=1, keepdims=True)
        cand = jnp.where(dd == mn, lanes, N)
        si = jnp.min(cand, axis=1, keepdims=True)
        return jnp.where(lanes == si, _BIG, dd)

    dfin = jax.lax.fori_loop(0, _KN, sel, d)
    mask = dfin >= 1e29

    for c in range(_DP):
        fc = fT[c:c + 1, :]
        hscr[:, c:c + 1] = jnp.max(jnp.where(mask, fc, -1.0), axis=1,
                                   keepdims=True)
    h = hscr[:]

    lanesA = jax.lax.broadcasted_iota(jnp.int32, (_KH * _M, _M), 1)
    inv_sqrt = jnp.float32(1.0) / jnp.sqrt(jnp.float32(_DL))
    dv = _DH // _KH

    for l in range(_LAYERS):
        q = _bdot(h, wqr[l])
        k = _bdot(h, wkr[l])
        v = _bdot(h, wvr[l])
        ss = [jax.lax.dot_general(
                  _bf(q[:, hd * _DL:(hd + 1) * _DL]),
                  _bf(k[:, hd * _DL:(hd + 1) * _DL]),
                  (((1,), (1,)), ((), ())),
                  preferred_element_type=jnp.float32)
              for hd in range(_KH)]
        s = jnp.concatenate(ss, axis=0) * inv_sqrt

        def asel(_, sc):
            mx = jnp.max(sc, axis=1, keepdims=True)
            cand = jnp.where(sc == mx, lanesA, _M)
            si = jnp.min(cand, axis=1, keepdims=True)
            return jnp.where(lanesA == si, _NEG, sc)

        sfin = jax.lax.fori_loop(0, _KA, asel, s)
        sm = jnp.where(sfin <= -1e29, s, _NEG)
        mx = jnp.max(sm, axis=1, keepdims=True)
        e = jnp.exp(sm - mx)
        p = e / jnp.sum(e, axis=1, keepdims=True)

        acc = jnp.zeros((_M, _DP), dtype=jnp.float32)
        for hd in range(_KH):
            ph = p[hd * _M:(hd + 1) * _M]
            vh = v[:, hd * dv:(hd + 1) * dv]
            oh_out = jnp.dot(ph, vh, preferred_element_type=jnp.float32,
                             precision=jax.lax.Precision.HIGHEST)
            acc = acc + _bdot(oh_out, wor[l][hd * dv:(hd + 1) * dv, :])
        h = h + acc

    g = jnp.max(h, axis=0, keepdims=True)
    logits = _bdot(g, whr[:]) + bhr[:]
    z = logits - jnp.max(logits, axis=1, keepdims=True)
    e = jnp.exp(z)
    outr[:] = (e / jnp.sum(e, axis=1, keepdims=True))[None]


def kernel(x, W_np, b_np, Wq, Wk, Wv, Wo, Wh, bh):
    B, N, _ = x.shape
    f32 = jnp.float32
    x0 = x[:, :, 0]
    x1 = x[:, :, 1]
    x2 = x[:, :, 2]

    kp0, kp1, kp2 = pl.pallas_call(
        _fps_body,
        out_shape=[jax.ShapeDtypeStruct((B, _M), f32)] * 3,
    )(x0, x1, x2)

    x0r = x0.reshape(B, 1, N)
    x1r = x1.reshape(B, 1, N)
    x2r = x2.reshape(B, 1, N)
    kp0r = kp0.reshape(B, _M, 1)
    kp1r = kp1.reshape(B, _M, 1)
    kp2r = kp2.reshape(B, _M, 1)
    wt = W_np.T
    bc = b_np.reshape(_DP, 1)
    bhr = bh.reshape(1, _NC)

    row = lambda i: (i, 0, 0)
    full2 = lambda a: pl.BlockSpec(a.shape, lambda i: (0, 0))
    full3 = lambda a: pl.BlockSpec(a.shape, lambda i: (0, 0, 0))

    out = pl.pallas_call(
        _main_body,
        grid=(B,),
        in_specs=[
            pl.BlockSpec((1, 1, N), row), pl.BlockSpec((1, 1, N), row),
            pl.BlockSpec((1, 1, N), row),
            pl.BlockSpec((1, _M, 1), row), pl.BlockSpec((1, _M, 1), row),
            pl.BlockSpec((1, _M, 1), row),
            full2(wt), full2(bc),
            full3(Wq), full3(Wk), full3(Wv), full3(Wo),
            full2(Wh), full2(bhr),
        ],
        out_specs=pl.BlockSpec((1, 1, _NC), row),
        out_shape=jax.ShapeDtypeStruct((B, 1, _NC), f32),
        scratch_shapes=[pltpu.VMEM((_M, _DP), f32)],
    )(x0r, x1r, x2r, kp0r, kp1r, kp2r, wt, bc, Wq, Wk, Wv, Wo, Wh, bhr)

    return out.reshape(B, _NC)
